# baseline (speedup 1.0000x reference)
import jax
import jax.numpy as jnp
from jax import lax
from jax.experimental import pallas as pl
from jax.experimental.pallas import tpu as pltpu
from jax.experimental.pallas import tpu_sc as plsc

B, L = 4096, 200
DW, DP = 32, 32
DO = DW + DP
N = B * L
NC, NS = 2, 16
NW = NC * NS
PER_W = N // NW
SG = 512
NSG = PER_W // SG
NBUF = 2


def _body(words_hbm, pos_hbm, ww_hbm, wp_hbm, out_hbm,
          idxw_v, idxp_v, rw_v, rp_v, semw, semp, semo):
    wid = lax.axis_index("s") * NC + lax.axis_index("c")
    pltpu.sync_copy(words_hbm.at[wid], idxw_v)
    pltpu.sync_copy(pos_hbm.at[wid], idxp_v)

    def start_gather(b, j):
        pltpu.async_copy(ww_hbm.at[idxw_v.at[j]], rw_v.at[b], semw.at[b])
        pltpu.async_copy(wp_hbm.at[idxp_v.at[j]], rp_v.at[b], semp.at[b])

    def wait_gather(b, j):
        pltpu.make_async_copy(ww_hbm.at[idxw_v.at[j]], rw_v.at[b],
                              semw.at[b]).wait()
        pltpu.make_async_copy(wp_hbm.at[idxp_v.at[j]], rp_v.at[b],
                              semp.at[b]).wait()

    base = wid * PER_W

    def start_write(b, j):
        row0 = base + j * SG
        pltpu.async_copy(rw_v.at[b],
                         out_hbm.at[pl.ds(row0, SG), pl.ds(0, DW)], semo.at[b])
        pltpu.async_copy(rp_v.at[b],
                         out_hbm.at[pl.ds(row0, SG), pl.ds(DW, DP)], semo.at[b])

    def wait_write(b, j):
        row0 = base + j * SG
        pltpu.make_async_copy(rw_v.at[b],
                              out_hbm.at[pl.ds(row0, SG), pl.ds(0, DW)],
                              semo.at[b]).wait()
        pltpu.make_async_copy(rp_v.at[b],
                              out_hbm.at[pl.ds(row0, SG), pl.ds(DW, DP)],
                              semo.at[b]).wait()

    for b in range(NBUF):
        start_gather(b, b)

    def step(it, carry):
        g = it * NBUF
        for b in range(NBUF):
            j = g + b
            wait_gather(b, j)
            start_write(b, j)
            wait_write(b, j)
            start_gather(b, j + NBUF)
        return carry

    lax.fori_loop(0, NSG // NBUF - 1, step, 0)

    for b in range(NBUF):
        j = NSG - NBUF + b
        wait_gather(b, j)
        start_write(b, j)
        wait_write(b, j)


@jax.jit
def _run(words_r, pos_r, W_words, W_pos):
    mesh = plsc.VectorSubcoreMesh(
        core_axis_name="c", subcore_axis_name="s",
        num_cores=NC, num_subcores=NS)
    f = pl.kernel(
        _body,
        out_type=jax.ShapeDtypeStruct((N, DO), jnp.float32),
        mesh=mesh,
        compiler_params=pltpu.CompilerParams(use_tc_tiling_on_sc=False),
        scratch_types=[
            pltpu.VMEM((NSG, SG), jnp.int32),
            pltpu.VMEM((NSG, SG), jnp.int32),
            pltpu.VMEM((NBUF, SG, DW), jnp.float32),
            pltpu.VMEM((NBUF, SG, DP), jnp.float32),
            pltpu.SemaphoreType.DMA((NBUF,)),
            pltpu.SemaphoreType.DMA((NBUF,)),
            pltpu.SemaphoreType.DMA((NBUF,)),
        ],
    )
    return f(words_r, pos_r, W_words, W_pos)


def kernel(words, pos, W_words, W_pos):
    words_r = words.astype(jnp.int32).reshape(NW, NSG, SG)
    pos_r = pos.astype(jnp.int32).reshape(NW, NSG, SG)
    out = _run(words_r, pos_r, W_words, W_pos)
    return out.reshape(B, L, DO)


# speedup vs baseline: 1.0010x; 1.0010x over previous
import jax
import jax.numpy as jnp
from jax import lax
from jax.experimental import pallas as pl
from jax.experimental.pallas import tpu as pltpu
from jax.experimental.pallas import tpu_sc as plsc

B, L = 4096, 200
DW, DP = 32, 32
DO = DW + DP
NC, NS = 2, 16
NW = NC * NS
BW = B // NW
KB = 2
NGR = BW // KB
NBUF = 2


def _body(words_hbm, pos_hbm, ww_hbm, wp_hbm, out_hbm,
          idxw_v, idxp_v, rw_v, rp_v, semw, semp, semo):
    wid = lax.axis_index("s") * NC + lax.axis_index("c")
    pltpu.sync_copy(words_hbm.at[wid], idxw_v)
    pltpu.sync_copy(pos_hbm.at[wid], idxp_v)
    b_base = wid * BW

    def start_gather(b, j):
        for k in range(KB):
            r = j * KB + k
            pltpu.async_copy(ww_hbm.at[idxw_v.at[r]], rw_v.at[b, k],
                             semw.at[b])
            pltpu.async_copy(wp_hbm.at[idxp_v.at[r]], rp_v.at[b, k],
                             semp.at[b])

    def wait_gather(b, j):
        for k in range(KB):
            r = j * KB + k
            pltpu.make_async_copy(ww_hbm.at[idxw_v.at[r]], rw_v.at[b, k],
                                  semw.at[b]).wait()
            pltpu.make_async_copy(wp_hbm.at[idxp_v.at[r]], rp_v.at[b, k],
                                  semp.at[b]).wait()

    def start_write(b, j):
        b0 = b_base + j * KB
        pltpu.async_copy(rw_v.at[b],
                         out_hbm.at[pl.ds(b0, KB), :, pl.ds(0, DW)],
                         semo.at[b])
        pltpu.async_copy(rp_v.at[b],
                         out_hbm.at[pl.ds(b0, KB), :, pl.ds(DW, DP)],
                         semo.at[b])

    def wait_write(b, j):
        b0 = b_base + j * KB
        pltpu.make_async_copy(rw_v.at[b],
                              out_hbm.at[pl.ds(b0, KB), :, pl.ds(0, DW)],
                              semo.at[b]).wait()
        pltpu.make_async_copy(rp_v.at[b],
                              out_hbm.at[pl.ds(b0, KB), :, pl.ds(DW, DP)],
                              semo.at[b]).wait()

    for b in range(NBUF):
        start_gather(b, b)

    def step(it, carry):
        g = it * NBUF
        for b in range(NBUF):
            j = g + b
            wait_gather(b, j)
            start_write(b, j)
            wait_write(b, j)
            start_gather(b, j + NBUF)
        return carry

    lax.fori_loop(0, NGR // NBUF - 1, step, 0)

    for b in range(NBUF):
        j = NGR - NBUF + b
        wait_gather(b, j)
        start_write(b, j)
        wait_write(b, j)


@jax.jit
def _run(words_r, pos_r, W_words, W_pos):
    mesh = plsc.VectorSubcoreMesh(
        core_axis_name="c", subcore_axis_name="s",
        num_cores=NC, num_subcores=NS)
    f = pl.kernel(
        _body,
        out_type=jax.ShapeDtypeStruct((B, L, DO), jnp.float32),
        mesh=mesh,
        compiler_params=pltpu.CompilerParams(use_tc_tiling_on_sc=False),
        scratch_types=[
            pltpu.VMEM((BW, L), jnp.int32),
            pltpu.VMEM((BW, L), jnp.int32),
            pltpu.VMEM((NBUF, KB, L, DW), jnp.float32),
            pltpu.VMEM((NBUF, KB, L, DP), jnp.float32),
            pltpu.SemaphoreType.DMA((NBUF,)),
            pltpu.SemaphoreType.DMA((NBUF,)),
            pltpu.SemaphoreType.DMA((NBUF,)),
        ],
    )
    return f(words_r, pos_r, W_words, W_pos)


def kernel(words, pos, W_words, W_pos):
    words_r = words.astype(jnp.int32).reshape(NW, BW, L)
    pos_r = pos.astype(jnp.int32).reshape(NW, BW, L)
    return _run(words_r, pos_r, W_words, W_pos)


# speedup vs baseline: 1.2932x; 1.2920x over previous
import jax
import jax.numpy as jnp
from jax import lax
from jax.experimental import pallas as pl
from jax.experimental.pallas import tpu as pltpu
from jax.experimental.pallas import tpu_sc as plsc

B, L = 4096, 200
DW, DP = 32, 32
DO = DW + DP
NC, NS = 2, 16
NW = NC * NS
BW = B // NW
KB = 2
NGR = BW // KB
NBUF = 2


def _body(words_hbm, pos_hbm, ww_hbm, wp_hbm, out_hbm,
          idxw_v, idxp_v, rw_v, rp_v, semw, semp, semo):
    wid = lax.axis_index("s") * NC + lax.axis_index("c")
    pltpu.sync_copy(words_hbm.at[wid], idxw_v)
    pltpu.sync_copy(pos_hbm.at[wid], idxp_v)
    b_base = wid * BW

    def start_gather(b, j):
        for k in range(KB):
            r = j * KB + k
            pltpu.async_copy(ww_hbm.at[idxw_v.at[r]], rw_v.at[b, k],
                             semw.at[b])
            pltpu.async_copy(wp_hbm.at[idxp_v.at[r]], rp_v.at[b, k],
                             semp.at[b])

    def wait_gather(b, j):
        for k in range(KB):
            r = j * KB + k
            pltpu.make_async_copy(ww_hbm.at[idxw_v.at[r]], rw_v.at[b, k],
                                  semw.at[b]).wait()
            pltpu.make_async_copy(wp_hbm.at[idxp_v.at[r]], rp_v.at[b, k],
                                  semp.at[b]).wait()

    def start_write(b, j):
        b0 = b_base + j * KB
        pltpu.async_copy(rw_v.at[b],
                         out_hbm.at[pl.ds(b0, KB), :, pl.ds(0, DW)],
                         semo.at[b])
        pltpu.async_copy(rp_v.at[b],
                         out_hbm.at[pl.ds(b0, KB), :, pl.ds(DW, DP)],
                         semo.at[b])

    def wait_write(b, j):
        b0 = b_base + j * KB
        pltpu.make_async_copy(rw_v.at[b],
                              out_hbm.at[pl.ds(b0, KB), :, pl.ds(0, DW)],
                              semo.at[b]).wait()
        pltpu.make_async_copy(rp_v.at[b],
                              out_hbm.at[pl.ds(b0, KB), :, pl.ds(DW, DP)],
                              semo.at[b]).wait()

    for b in range(NBUF):
        start_gather(b, b)

    def step(it, carry):
        g = it * NBUF
        for b in range(NBUF):
            j = g + b
            wait_gather(b, j)
            start_write(b, j)
            wait_write(b, j)
            start_gather(b, j + NBUF)
        return carry

    lax.fori_loop(0, NGR // NBUF - 1, step, 0)

    for b in range(NBUF):
        j = NGR - NBUF + b
        wait_gather(b, j)
        start_write(b, j)
        wait_write(b, j)


@jax.jit
def _run(words_r, pos_r, W_words, W_pos):
    mesh = plsc.VectorSubcoreMesh(
        core_axis_name="c", subcore_axis_name="s",
        num_cores=NC, num_subcores=NS)
    f = pl.kernel(
        _body,
        out_type=jax.ShapeDtypeStruct((B, L, 128), jnp.float32),
        mesh=mesh,
        compiler_params=pltpu.CompilerParams(use_tc_tiling_on_sc=False),
        scratch_types=[
            pltpu.VMEM((BW, L), jnp.int32),
            pltpu.VMEM((BW, L), jnp.int32),
            pltpu.VMEM((NBUF, KB, L, DW), jnp.float32),
            pltpu.VMEM((NBUF, KB, L, DP), jnp.float32),
            pltpu.SemaphoreType.DMA((NBUF,)),
            pltpu.SemaphoreType.DMA((NBUF,)),
            pltpu.SemaphoreType.DMA((NBUF,)),
        ],
    )
    return f(words_r, pos_r, W_words, W_pos)


def kernel(words, pos, W_words, W_pos):
    words_r = words.astype(jnp.int32).reshape(NW, BW, L)
    pos_r = pos.astype(jnp.int32).reshape(NW, BW, L)
    out = _run(words_r, pos_r, W_words, W_pos)
    return out[:, :, :DO]


# speedup vs baseline: 1.3048x; 1.0090x over previous
import jax
import jax.numpy as jnp
from jax import lax
from jax.experimental import pallas as pl
from jax.experimental.pallas import tpu as pltpu
from jax.experimental.pallas import tpu_sc as plsc

B, L = 4096, 200
DW, DP = 32, 32
DO = DW + DP
N = B * L
NC, NS = 2, 16
NW = NC * NS
BW = B // NW
PER_W = BW * L
KB = 2
NGR = BW // KB
NBUF = 2
VLANES = 16


def _body(words_hbm, pos_hbm, ww_hbm, wp_hbm, out_hbm,
          idxw_v, idxp_v, rw_v, rp_v, semw, semp, semo):
    wid = lax.axis_index("s") * NC + lax.axis_index("c")
    base = wid * PER_W
    pltpu.sync_copy(words_hbm.at[pl.ds(base, PER_W)], idxw_v)
    pltpu.sync_copy(pos_hbm.at[pl.ds(base, PER_W)], idxp_v)
    b_base = wid * BW

    def scale(i, carry):
        s = pl.ds(i * VLANES, VLANES)
        idxw_v[s] = idxw_v[s] * 4
        return carry

    lax.fori_loop(0, PER_W // VLANES, scale, 0)

    def start_gather(b, j):
        for k in range(KB):
            r = j * KB + k
            pltpu.async_copy(ww_hbm.at[idxw_v.at[pl.ds(r * L, L)]],
                             rw_v.at[b, k], semw.at[b])
            pltpu.async_copy(wp_hbm.at[idxp_v.at[pl.ds(r * L, L)]],
                             rp_v.at[b, k], semp.at[b])

    def wait_gather(b, j):
        for k in range(KB):
            r = j * KB + k
            pltpu.make_async_copy(ww_hbm.at[idxw_v.at[pl.ds(r * L, L)]],
                                  rw_v.at[b, k], semw.at[b]).wait()
            pltpu.make_async_copy(wp_hbm.at[idxp_v.at[pl.ds(r * L, L)]],
                                  rp_v.at[b, k], semp.at[b]).wait()

    def start_write(b, j):
        b0 = b_base + j * KB
        pltpu.async_copy(rw_v.at[b],
                         out_hbm.at[pl.ds(b0, KB), :, pl.ds(0, DW)],
                         semo.at[b])
        pltpu.async_copy(rp_v.at[b],
                         out_hbm.at[pl.ds(b0, KB), :, pl.ds(DW, DP)],
                         semo.at[b])

    def wait_write(b, j):
        b0 = b_base + j * KB
        pltpu.make_async_copy(rw_v.at[b],
                              out_hbm.at[pl.ds(b0, KB), :, pl.ds(0, DW)],
                              semo.at[b]).wait()
        pltpu.make_async_copy(rp_v.at[b],
                              out_hbm.at[pl.ds(b0, KB), :, pl.ds(DW, DP)],
                              semo.at[b]).wait()

    for b in range(NBUF):
        start_gather(b, b)

    def step(it, carry):
        g = it * NBUF
        for b in range(NBUF):
            j = g + b
            wait_gather(b, j)
            start_write(b, j)
            wait_write(b, j)
            start_gather(b, j + NBUF)
        return carry

    lax.fori_loop(0, NGR // NBUF - 1, step, 0)

    for b in range(NBUF):
        j = NGR - NBUF + b
        wait_gather(b, j)
        start_write(b, j)
        wait_write(b, j)


@jax.jit
def _run(words_f, pos_f, W4, W_pos):
    mesh = plsc.VectorSubcoreMesh(
        core_axis_name="c", subcore_axis_name="s",
        num_cores=NC, num_subcores=NS)
    f = pl.kernel(
        _body,
        out_type=jax.ShapeDtypeStruct((B, L, 128), jnp.float32),
        mesh=mesh,
        compiler_params=pltpu.CompilerParams(use_tc_tiling_on_sc=False),
        scratch_types=[
            pltpu.VMEM((PER_W,), jnp.int32),
            pltpu.VMEM((PER_W,), jnp.int32),
            pltpu.VMEM((NBUF, KB, L, DW), jnp.float32),
            pltpu.VMEM((NBUF, KB, L, DP), jnp.float32),
            pltpu.SemaphoreType.DMA((NBUF,)),
            pltpu.SemaphoreType.DMA((NBUF,)),
            pltpu.SemaphoreType.DMA((NBUF,)),
        ],
    )
    return f(words_f, pos_f, W4, W_pos)


def kernel(words, pos, W_words, W_pos):
    W4 = jnp.pad(W_words, ((0, 0), (0, 128 - DW))).reshape(4 * 1000000, DW)
    words_f = words.astype(jnp.int32).reshape(N)
    pos_f = pos.astype(jnp.int32).reshape(N)
    out = _run(words_f, pos_f, W4, W_pos)
    return out[:, :, :DO]
